# baseline (speedup 1.0000x reference)
import functools

import jax
import jax.numpy as jnp
from jax import lax
from jax.experimental import pallas as pl
from jax.experimental.pallas import tpu as pltpu
from jax.experimental.pallas import tpu_sc as plsc

_B = 4096
_F = 26
_D = 64
_V = 100000
_N = _F * _V
_L = 16
_CHK = 12544
_HALF = _CHK // 2
_TAIL0 = 2599936
_LAST7 = _TAIL0 - _CHK
_LO7 = (2499968 + 7 * _CHK) - _LAST7


def _body(x3_hbm, wt_hbm, bias_hbm, tail4_hbm, out5_hbm, part_hbm,
          xv0, xv1, bufa, bufb, acc, macc, tailv, biasv,
          sema, semb, semx, semp):
    cid = lax.axis_index("c")
    sid = lax.axis_index("s")
    k = sid % 8
    bh = sid // 8

    pltpu.sync_copy(bias_hbm, biasv)
    xrow = pl.multiple_of(bh * 16, 8)

    zero16 = jnp.zeros((_L,), jnp.float32)

    def choff_of(f):
        delta = 32 * (f % 4)
        base = f * _V - delta + k * _CHK
        return pl.multiple_of(
            jnp.where((f == _F - 1) & (k == 7), _LAST7, base), 128)

    for _pb in (bufa, bufb):
        for _r in range(8):
            _pb[_r, pl.ds(_HALF, _L)] = zero16

    def tr_body(tr, tr_carry):
        trg = 4 * cid + tr
        row0 = pl.multiple_of(8 * trg, 8)

        @plsc.parallel_loop(0, 16)
        def zblk(p):
            for qi in range(8):
                q0 = qi * _L
                for r in range(8):
                    acc[r, p, pl.ds(q0, _L)] = zero16

        def issue(f, h, buf, sem):
            choff = pl.multiple_of(choff_of(f) + h * _HALF, 128)
            pltpu.async_copy(
                wt_hbm.at[pl.ds(row0, 8), pl.ds(choff, _HALF)],
                buf.at[:, pl.ds(0, _HALF)], sem)

        def issue_x(f, xv, sem):
            pltpu.async_copy(
                x3_hbm.at[f, pl.ds(xrow, 16)], xv, sem)

        def wait(buf, sem):
            pltpu.make_async_copy(
                wt_hbm.at[pl.ds(0, 8), pl.ds(0, _HALF)],
                buf.at[:, pl.ds(0, _HALF)], sem).wait()

        def wait_x(xv, sem):
            pltpu.make_async_copy(
                x3_hbm.at[0, pl.ds(0, 16)], xv, sem).wait()

        def scan(f, h, buf, xv, lo):
            shift = f * _V - (choff_of(f) + h * _HALF)

            @plsc.parallel_loop(0, 16, unroll=2)
            def prow(p):
                for qi in range(8):
                    q0 = qi * _L
                    xa = xv[p, pl.ds(q0, _L)]
                    idm = jnp.minimum(
                        (xa + (shift - lo)).astype(jnp.uint32),
                        jnp.uint32(_HALF) - lo.astype(jnp.uint32))
                    idc = idm.astype(jnp.int32) + lo
                    for r in range(8):
                        rvec = jnp.full((_L,), r, dtype=jnp.int32)
                        v = plsc.load_gather(buf, [rvec, idc])
                        plsc.addupdate(acc.at[r, p, pl.ds(q0, _L)], v)

        issue_x(0, xv0, semx)
        issue(0, 0, bufa, sema)
        issue(0, 1, bufb, semb)
        wait_x(xv0, semx)

        def f2_body(f2, carry):
            for ff in range(2):
                f = 2 * f2 + ff
                xv = xv0 if ff == 0 else xv1
                xvn = xv1 if ff == 0 else xv0
                lo0 = jnp.where((f == _F - 1) & (k == 7), _LO7, 0)

                @pl.when(f < _F - 1)
                def _():
                    issue_x(f + 1, xvn, semx)

                wait(bufa, sema)
                scan(f, 0, bufa, xv, lo0)

                @pl.when(f < _F - 1)
                def _():
                    issue(f + 1, 0, bufa, sema)

                wait(bufb, semb)
                scan(f, 1, bufb, xv, jnp.int32(0))

                @pl.when(f < _F - 1)
                def _():
                    issue(f + 1, 1, bufb, semb)
                    wait_x(xvn, semx)

            return carry

        lax.fori_loop(0, _F // 2, f2_body, 0)

        @pl.when(k == 0)
        def _():
            pltpu.sync_copy(tail4_hbm.at[trg], tailv)

            @plsc.parallel_loop(0, 16)
            def blkt(p):
                for qi in range(8):
                    q0 = qi * _L
                    xa = xv1[p, pl.ds(q0, _L)]
                    idxl = xa - (_TAIL0 - (_F - 1) * _V)
                    valid = idxl >= 0
                    idc = jnp.clip(idxl, 0, _N - _TAIL0 - 1)
                    for r in range(8):
                        rvec = jnp.full((_L,), r, dtype=jnp.int32)
                        v = plsc.load_gather(tailv, [rvec, idc])
                        vm = jnp.where(valid, v, 0.0)
                        plsc.addupdate(acc.at[r, p, pl.ds(q0, _L)], vm)

        for r in range(8):
            pltpu.async_copy(acc.at[r], part_hbm.at[trg, bh, r, k], semp)
        pltpu.make_async_copy(part_hbm.at[trg, bh, 0], acc, semp).wait()
        plsc.subcore_barrier()

        pltpu.sync_copy(part_hbm.at[trg, bh, k], acc)
        c = 8 * trg + k
        cvec = jnp.full((_L,), c, dtype=jnp.int32)
        bval = plsc.load_gather(biasv, [cvec])

        @plsc.parallel_loop(0, 16)
        def merge(p):
            for qi in range(8):
                q0 = qi * _L
                sl = pl.ds(q0, _L)
                s = acc[0, p, sl] + bval
                for kk in range(1, 8):
                    s = s + acc[kk, p, sl]
                macc[p, sl] = s
        pltpu.sync_copy(macc, out5_hbm.at[trg, k, bh])
        plsc.subcore_barrier()
        return tr_carry

    lax.fori_loop(0, 4, tr_body, 0)


@jax.jit
def _fieldlinear_sc(x3, wt, bias, tail4):
    mesh = plsc.VectorSubcoreMesh(core_axis_name="c", subcore_axis_name="s")
    kern = functools.partial(
        pl.kernel,
        out_type=(
            jax.ShapeDtypeStruct((8, 8, 2, 16, 128), jnp.float32),
            jax.ShapeDtypeStruct((8, 2, 8, 8, 16, 128), jnp.float32),
        ),
        mesh=mesh,
        compiler_params=pltpu.CompilerParams(needs_layout_passes=False),
        scratch_types=[
            pltpu.VMEM((16, 128), jnp.int32),
            pltpu.VMEM((16, 128), jnp.int32),
            pltpu.VMEM((8, _HALF + 128), jnp.float32),
            pltpu.VMEM((8, _HALF + 128), jnp.float32),
            pltpu.VMEM((8, 16, 128), jnp.float32),
            pltpu.VMEM((16, 128), jnp.float32),
            pltpu.VMEM((8, 64), jnp.float32),
            pltpu.VMEM((_D,), jnp.float32),
            pltpu.SemaphoreType.DMA,
            pltpu.SemaphoreType.DMA,
            pltpu.SemaphoreType.DMA,
            pltpu.SemaphoreType.DMA,
        ],
    )(_body)
    return kern(x3, wt, bias, tail4)


def kernel(x, weight, bias):
    x3 = x.T.reshape(_F, 32, 128)
    tail4 = weight[_TAIL0:].T.reshape(8, 8, _N - _TAIL0)
    o5, _part = _fieldlinear_sc(x3, weight.T, bias, tail4)
    return o5.reshape(_D, _B).T


# speedup vs baseline: 1.1669x; 1.1669x over previous
import functools

import jax
import jax.numpy as jnp
from jax import lax
from jax.experimental import pallas as pl
from jax.experimental.pallas import tpu as pltpu
from jax.experimental.pallas import tpu_sc as plsc

_B = 4096
_F = 26
_D = 64
_V = 100000
_N = _F * _V
_L = 16
_CHK = 12544
_HALF = _CHK // 2
_TAIL0 = 2599936
_LAST7 = _TAIL0 - _CHK
_LO7 = (2499968 + 7 * _CHK) - _LAST7


def _body(x3_hbm, wt_hbm, bias_hbm, tail4_hbm, out5_hbm, part_hbm,
          xv0, xv1, bufa, bufb, acc, macc, tailv, biasv,
          sema, semb, semx, semp):
    cid = lax.axis_index("c")
    sid = lax.axis_index("s")
    k = sid % 8
    bh = sid // 8

    pltpu.sync_copy(bias_hbm, biasv)
    xrow = pl.multiple_of(bh * 16, 8)

    zero16 = jnp.zeros((_L,), jnp.float32)

    def choff_of(f):
        delta = 32 * (f % 4)
        base = f * _V - delta + k * _CHK
        return pl.multiple_of(
            jnp.where((f == _F - 1) & (k == 7), _LAST7, base), 128)

    for _pb in (bufa, bufb):
        for _r in range(8):
            _pb[_r, pl.ds(_HALF, _L)] = zero16

    def tr_body(tr, tr_carry):
        trg = 4 * cid + tr
        row0 = pl.multiple_of(8 * trg, 8)

        @plsc.parallel_loop(0, 16)
        def zblk(p):
            for qi in range(8):
                q0 = qi * _L
                for r in range(8):
                    acc[r, p, pl.ds(q0, _L)] = zero16

        def issue(f, h, buf, sem):
            choff = pl.multiple_of(choff_of(f) + h * _HALF, 128)
            pltpu.async_copy(
                wt_hbm.at[pl.ds(row0, 8), pl.ds(choff, _HALF)],
                buf.at[:, pl.ds(0, _HALF)], sem)

        def issue_x(f, xv, sem):
            pltpu.async_copy(
                x3_hbm.at[f, pl.ds(xrow, 16)], xv, sem)

        def wait(buf, sem):
            pltpu.make_async_copy(
                wt_hbm.at[pl.ds(0, 8), pl.ds(0, _HALF)],
                buf.at[:, pl.ds(0, _HALF)], sem).wait()

        def wait_x(xv, sem):
            pltpu.make_async_copy(
                x3_hbm.at[0, pl.ds(0, 16)], xv, sem).wait()

        def scan(f, h, buf, xv, lo):
            shift = f * _V - (choff_of(f) + h * _HALF)

            @plsc.parallel_loop(0, 32)
            def prow(i):
                p = i // 2
                for qi2 in range(4):
                    qi = (i % 2) * 4 + qi2
                    q0 = qi * _L
                    xa = xv[p, pl.ds(q0, _L)]
                    idm = jnp.minimum(
                        (xa + (shift - lo)).astype(jnp.uint32),
                        jnp.uint32(_HALF) - lo.astype(jnp.uint32))
                    idc = idm.astype(jnp.int32) + lo
                    for r in range(8):
                        rvec = jnp.full((_L,), r, dtype=jnp.int32)
                        v = plsc.load_gather(buf, [rvec, idc])
                        plsc.addupdate(acc.at[r, p, pl.ds(q0, _L)], v)

        issue_x(0, xv0, semx)
        issue(0, 0, bufa, sema)
        issue(0, 1, bufb, semb)
        wait_x(xv0, semx)

        def f2_body(f2, carry):
            for ff in range(2):
                f = 2 * f2 + ff
                xv = xv0 if ff == 0 else xv1
                xvn = xv1 if ff == 0 else xv0
                lo0 = jnp.where((f == _F - 1) & (k == 7), _LO7, 0)

                @pl.when(f < _F - 1)
                def _():
                    issue_x(f + 1, xvn, semx)

                wait(bufa, sema)
                scan(f, 0, bufa, xv, lo0)

                @pl.when(f < _F - 1)
                def _():
                    issue(f + 1, 0, bufa, sema)

                wait(bufb, semb)
                scan(f, 1, bufb, xv, jnp.int32(0))

                @pl.when(f < _F - 1)
                def _():
                    issue(f + 1, 1, bufb, semb)
                    wait_x(xvn, semx)

            return carry

        lax.fori_loop(0, _F // 2, f2_body, 0)

        @pl.when(k == 0)
        def _():
            pltpu.sync_copy(tail4_hbm.at[trg], tailv)

            @plsc.parallel_loop(0, 16)
            def blkt(p):
                for qi in range(8):
                    q0 = qi * _L
                    xa = xv1[p, pl.ds(q0, _L)]
                    idxl = xa - (_TAIL0 - (_F - 1) * _V)
                    valid = idxl >= 0
                    idc = jnp.clip(idxl, 0, _N - _TAIL0 - 1)
                    for r in range(8):
                        rvec = jnp.full((_L,), r, dtype=jnp.int32)
                        v = plsc.load_gather(tailv, [rvec, idc])
                        vm = jnp.where(valid, v, 0.0)
                        plsc.addupdate(acc.at[r, p, pl.ds(q0, _L)], vm)

        for r in range(8):
            pltpu.async_copy(acc.at[r], part_hbm.at[trg, bh, r, k], semp)
        pltpu.make_async_copy(part_hbm.at[trg, bh, 0], acc, semp).wait()
        plsc.subcore_barrier()

        pltpu.sync_copy(part_hbm.at[trg, bh, k], acc)
        c = 8 * trg + k
        cvec = jnp.full((_L,), c, dtype=jnp.int32)
        bval = plsc.load_gather(biasv, [cvec])

        @plsc.parallel_loop(0, 16)
        def merge(p):
            for qi in range(8):
                q0 = qi * _L
                sl = pl.ds(q0, _L)
                s = acc[0, p, sl] + bval
                for kk in range(1, 8):
                    s = s + acc[kk, p, sl]
                macc[p, sl] = s
        pltpu.sync_copy(macc, out5_hbm.at[trg, k, bh])
        plsc.subcore_barrier()
        return tr_carry

    lax.fori_loop(0, 4, tr_body, 0)


@jax.jit
def _fieldlinear_sc(x3, wt, bias, tail4):
    mesh = plsc.VectorSubcoreMesh(core_axis_name="c", subcore_axis_name="s")
    kern = functools.partial(
        pl.kernel,
        out_type=(
            jax.ShapeDtypeStruct((8, 8, 2, 16, 128), jnp.float32),
            jax.ShapeDtypeStruct((8, 2, 8, 8, 16, 128), jnp.float32),
        ),
        mesh=mesh,
        compiler_params=pltpu.CompilerParams(needs_layout_passes=False),
        scratch_types=[
            pltpu.VMEM((16, 128), jnp.int32),
            pltpu.VMEM((16, 128), jnp.int32),
            pltpu.VMEM((8, _HALF + 128), jnp.float32),
            pltpu.VMEM((8, _HALF + 128), jnp.float32),
            pltpu.VMEM((8, 16, 128), jnp.float32),
            pltpu.VMEM((16, 128), jnp.float32),
            pltpu.VMEM((8, 64), jnp.float32),
            pltpu.VMEM((_D,), jnp.float32),
            pltpu.SemaphoreType.DMA,
            pltpu.SemaphoreType.DMA,
            pltpu.SemaphoreType.DMA,
            pltpu.SemaphoreType.DMA,
        ],
    )(_body)
    return kern(x3, wt, bias, tail4)


def kernel(x, weight, bias):
    x3 = x.T.reshape(_F, 32, 128)
    tail4 = weight[_TAIL0:].T.reshape(8, 8, _N - _TAIL0)
    o5, _part = _fieldlinear_sc(x3, weight.T, bias, tail4)
    return o5.reshape(_D, _B).T


# speedup vs baseline: 1.3116x; 1.1240x over previous
import functools

import jax
import jax.numpy as jnp
from jax import lax
from jax.experimental import pallas as pl
from jax.experimental.pallas import tpu as pltpu
from jax.experimental.pallas import tpu_sc as plsc

_B = 4096
_F = 26
_D = 64
_V = 100000
_N = _F * _V
_L = 16
_CHK = 12544
_HALF = _CHK // 2
_TAIL0 = 2599936
_LAST7 = _TAIL0 - _CHK
_LO7 = (2499968 + 7 * _CHK) - _LAST7


def _body(x3_hbm, wt_hbm, bias_hbm, tail4_hbm, out5_hbm, part_hbm,
          xv0, xv1, bufa, bufb, acc, macc, tailv, biasv,
          sema, semb, semx, semp):
    cid = lax.axis_index("c")
    sid = lax.axis_index("s")
    k = sid % 8
    bh = sid // 8

    pltpu.sync_copy(bias_hbm, biasv)
    xrow = pl.multiple_of(bh * 16, 8)

    zero16 = jnp.zeros((_L,), jnp.float32)

    def choff_of(f):
        delta = 32 * (f % 4)
        base = f * _V - delta + k * _CHK
        return pl.multiple_of(
            jnp.where((f == _F - 1) & (k == 7), _LAST7, base), 128)

    for _pb in (bufa, bufb):
        for _r in range(8):
            _pb[_r, pl.ds(_HALF, _L)] = zero16

    def tr_body(tr, tr_carry):
        trg = 4 * cid + tr
        row0 = pl.multiple_of(8 * trg, 8)

        @plsc.parallel_loop(0, 16)
        def zblk(p):
            for qi in range(8):
                q0 = qi * _L
                for r in range(8):
                    acc[r, p, pl.ds(q0, _L)] = zero16

        def issue(f, h, buf, sem):
            choff = pl.multiple_of(choff_of(f) + h * _HALF, 128)
            pltpu.async_copy(
                wt_hbm.at[pl.ds(row0, 8), pl.ds(choff, _HALF)],
                buf.at[:, pl.ds(0, _HALF)], sem)

        def issue_x(f, xv, sem):
            pltpu.async_copy(
                x3_hbm.at[f, pl.ds(xrow, 16)], xv, sem)

        def wait(buf, sem):
            pltpu.make_async_copy(
                wt_hbm.at[pl.ds(0, 8), pl.ds(0, _HALF)],
                buf.at[:, pl.ds(0, _HALF)], sem).wait()

        def wait_x(xv, sem):
            pltpu.make_async_copy(
                x3_hbm.at[0, pl.ds(0, 16)], xv, sem).wait()

        def scan(f, h, buf, xv, lo):
            shift = f * _V - (choff_of(f) + h * _HALF)

            @plsc.parallel_loop(0, 16)
            def prow(p):
                for qi in range(8):
                    q0 = qi * _L
                    xa = xv[p, pl.ds(q0, _L)]
                    idm = jnp.minimum(
                        (xa + (shift - lo)).astype(jnp.uint32),
                        jnp.uint32(_HALF) - lo.astype(jnp.uint32))
                    idc = idm.astype(jnp.int32) + lo
                    for r in range(8):
                        rvec = jnp.full((_L,), r, dtype=jnp.int32)
                        v = plsc.load_gather(buf, [rvec, idc])
                        plsc.addupdate(acc.at[r, p, pl.ds(q0, _L)], v)

        issue_x(0, xv0, semx)
        issue(0, 0, bufa, sema)
        issue(0, 1, bufb, semb)
        wait_x(xv0, semx)

        def f2_body(f2, carry):
            for ff in range(2):
                f = 2 * f2 + ff
                xv = xv0 if ff == 0 else xv1
                xvn = xv1 if ff == 0 else xv0
                lo0 = jnp.where((f == _F - 1) & (k == 7), _LO7, 0)

                @pl.when(f < _F - 1)
                def _():
                    issue_x(f + 1, xvn, semx)

                wait(bufa, sema)
                scan(f, 0, bufa, xv, lo0)

                @pl.when(f < _F - 1)
                def _():
                    issue(f + 1, 0, bufa, sema)

                wait(bufb, semb)
                scan(f, 1, bufb, xv, jnp.int32(0))

                @pl.when(f < _F - 1)
                def _():
                    issue(f + 1, 1, bufb, semb)
                    wait_x(xvn, semx)

            return carry

        lax.fori_loop(0, _F // 2, f2_body, 0)

        @pl.when(k == 0)
        def _():
            pltpu.sync_copy(tail4_hbm.at[trg], tailv)

            @plsc.parallel_loop(0, 16)
            def blkt(p):
                for qi in range(8):
                    q0 = qi * _L
                    xa = xv1[p, pl.ds(q0, _L)]
                    idxl = xa - (_TAIL0 - (_F - 1) * _V)
                    valid = idxl >= 0
                    idc = jnp.clip(idxl, 0, _N - _TAIL0 - 1)
                    for r in range(8):
                        rvec = jnp.full((_L,), r, dtype=jnp.int32)
                        v = plsc.load_gather(tailv, [rvec, idc])
                        vm = jnp.where(valid, v, 0.0)
                        plsc.addupdate(acc.at[r, p, pl.ds(q0, _L)], vm)

        for r in range(8):
            pltpu.async_copy(acc.at[r], part_hbm.at[trg, bh, r, k], semp)
        pltpu.make_async_copy(part_hbm.at[trg, bh, 0], acc, semp).wait()
        plsc.subcore_barrier()

        pltpu.sync_copy(part_hbm.at[trg, bh, k], acc)
        c = 8 * trg + k
        cvec = jnp.full((_L,), c, dtype=jnp.int32)
        bval = plsc.load_gather(biasv, [cvec])

        @plsc.parallel_loop(0, 16)
        def merge(p):
            for qi in range(8):
                q0 = qi * _L
                sl = pl.ds(q0, _L)
                s = acc[0, p, sl] + bval
                for kk in range(1, 8):
                    s = s + acc[kk, p, sl]
                macc[p, sl] = s
        pltpu.sync_copy(macc, out5_hbm.at[trg, k, bh])
        plsc.subcore_barrier()
        return tr_carry

    lax.fori_loop(0, 4, tr_body, 0)


@jax.jit
def _fieldlinear_sc(x3, wt, bias, tail4):
    mesh = plsc.VectorSubcoreMesh(core_axis_name="c", subcore_axis_name="s")
    kern = functools.partial(
        pl.kernel,
        out_type=(
            jax.ShapeDtypeStruct((8, 8, 2, 16, 128), jnp.float32),
            jax.ShapeDtypeStruct((8, 2, 8, 8, 16, 128), jnp.float32),
        ),
        mesh=mesh,
        compiler_params=pltpu.CompilerParams(needs_layout_passes=False),
        scratch_types=[
            pltpu.VMEM((16, 128), jnp.int32),
            pltpu.VMEM((16, 128), jnp.int32),
            pltpu.VMEM((8, _HALF + 128), jnp.float32),
            pltpu.VMEM((8, _HALF + 128), jnp.float32),
            pltpu.VMEM((8, 16, 128), jnp.float32),
            pltpu.VMEM((16, 128), jnp.float32),
            pltpu.VMEM((8, 64), jnp.float32),
            pltpu.VMEM((_D,), jnp.float32),
            pltpu.SemaphoreType.DMA,
            pltpu.SemaphoreType.DMA,
            pltpu.SemaphoreType.DMA,
            pltpu.SemaphoreType.DMA,
        ],
    )(_body)
    return kern(x3, wt, bias, tail4)


def kernel(x, weight, bias):
    x3 = x.T.reshape(_F, 32, 128)
    tail4 = weight[_TAIL0:].T.reshape(8, 8, _N - _TAIL0)
    o5, _part = _fieldlinear_sc(x3, weight.T, bias, tail4)
    return o5.reshape(_D, _B).T
